# baseline (speedup 1.0000x reference)
import jax
import jax.numpy as jnp
from jax.experimental import pallas as pl

_ROWS_PER_BLOCK = 64


def _tc_body(x_ref, wp_ref, o_ref):
    b = x_ref.shape[0]
    m = x_ref.shape[1]
    x = x_ref[...].astype(jnp.int32)
    wp = wp_ref[...]
    lane16 = jax.lax.broadcasted_iota(jnp.int32, (b, 16), 1)
    lane8 = lane16 & 7
    half = lane16 >> 3
    xlo = x[:, :128]
    xhi = x[:, 128:]
    for c in range(m // 2):
        if 2 * c + 1 < 128:
            idx = half + (2 * c)
            xc = jnp.take_along_axis(xlo, idx, axis=1)
        else:
            idx = half + (2 * c - 128)
            xc = jnp.take_along_axis(xhi, idx, axis=1)
        e2 = jnp.where(xc == lane8, 1.0, 0.0)
        outc = jax.lax.dot_general(
            e2, wp, (((1,), (0,)), ((), ())),
            preferred_element_type=jnp.float32)
        o_ref[:, 128 * c:128 * (c + 1)] = outc


def kernel(X, W):
    n, m = X.shape
    f = W.shape[1]
    mf = m * f
    b = _ROWS_PER_BLOCK
    wp = jnp.zeros((16, 2 * f), jnp.float32)
    wp = wp.at[:5, :f].set(W).at[8:13, f:].set(W)
    out2d = pl.pallas_call(
        _tc_body,
        grid=(n // b,),
        in_specs=[
            pl.BlockSpec((b, m), lambda i: (i, 0)),
            pl.BlockSpec((16, 2 * f), lambda i: (0, 0)),
        ],
        out_specs=pl.BlockSpec((b, mf), lambda i: (i, 0)),
        out_shape=jax.ShapeDtypeStruct((n, mf), jnp.float32),
    )(X.astype(jnp.int32), wp)
    return out2d.reshape(n, m, f)


# speedup vs baseline: 1.0944x; 1.0944x over previous
import jax
import jax.numpy as jnp
from jax.experimental import pallas as pl

_ROWS_PER_BLOCK = 512


def _tc_body(x_ref, wp_ref, o_ref):
    b = x_ref.shape[0]
    m = x_ref.shape[1]
    x = x_ref[...].astype(jnp.int32)
    wp = wp_ref[...]
    lane16 = jax.lax.broadcasted_iota(jnp.int32, (b, 16), 1)
    lane8 = lane16 & 7
    half = lane16 >> 3
    xlo = x[:, :128]
    xhi = x[:, 128:]
    for c in range(m // 2):
        if 2 * c + 1 < 128:
            idx = half + (2 * c)
            xc = jnp.take_along_axis(xlo, idx, axis=1)
        else:
            idx = half + (2 * c - 128)
            xc = jnp.take_along_axis(xhi, idx, axis=1)
        e2 = jnp.where(xc == lane8, 1.0, 0.0)
        outc = jax.lax.dot_general(
            e2, wp, (((1,), (0,)), ((), ())),
            preferred_element_type=jnp.float32)
        o_ref[:, 128 * c:128 * (c + 1)] = outc


def kernel(X, W):
    n, m = X.shape
    f = W.shape[1]
    mf = m * f
    b = _ROWS_PER_BLOCK
    wp = jnp.zeros((16, 2 * f), jnp.float32)
    wp = wp.at[:5, :f].set(W).at[8:13, f:].set(W)
    out2d = pl.pallas_call(
        _tc_body,
        grid=(n // b,),
        in_specs=[
            pl.BlockSpec((b, m), lambda i: (i, 0)),
            pl.BlockSpec((16, 2 * f), lambda i: (0, 0)),
        ],
        out_specs=pl.BlockSpec((b, mf), lambda i: (i, 0)),
        out_shape=jax.ShapeDtypeStruct((n, mf), jnp.float32),
    )(X.astype(jnp.int32), wp)
    return out2d.reshape(n, m, f)


# speedup vs baseline: 1.0951x; 1.0006x over previous
import jax
import jax.numpy as jnp
from jax.experimental import pallas as pl
from jax.experimental.pallas import tpu as pltpu

_ROWS_PER_BLOCK = 256
_SPLIT = 4


def _tc_body(x_ref, wp_ref, o_ref, acc_ref, sem_ref):
    b = x_ref.shape[0]
    m = x_ref.shape[1]
    i = pl.program_id(0)
    ni = pl.num_programs(0)
    slot = jax.lax.rem(i, 2)
    bs = b // _SPLIT

    def copy(s, k, row0):
        return pltpu.make_async_copy(
            acc_ref.at[s, pl.ds(k * bs, bs), :],
            o_ref.at[pl.ds(row0 + k * bs, bs), :],
            sem_ref.at[s, k])

    @pl.when(i >= 2)
    def _():
        for k in range(_SPLIT):
            copy(slot, k, (i - 2) * b).wait()

    x = x_ref[...].astype(jnp.int32)
    wp = wp_ref[...]
    lane16 = jax.lax.broadcasted_iota(jnp.int32, (b, 16), 1)
    lane8 = lane16 & 7
    half = lane16 >> 3
    xlo = x[:, :128]
    xhi = x[:, 128:]
    for c in range(m // 2):
        if 2 * c + 1 < 128:
            idx = half + (2 * c)
            xc = jnp.take_along_axis(xlo, idx, axis=1)
        else:
            idx = half + (2 * c - 128)
            xc = jnp.take_along_axis(xhi, idx, axis=1)
        e2 = jnp.where(xc == lane8, 1.0, 0.0)
        outc = jax.lax.dot_general(
            e2, wp, (((1,), (0,)), ((), ())),
            preferred_element_type=jnp.float32)
        acc_ref[slot, :, pl.ds(128 * c, 128)] = outc

    for k in range(_SPLIT):
        copy(slot, k, i * b).start()

    @pl.when(i == ni - 1)
    def _():
        for k in range(_SPLIT):
            copy(1 - slot, k, (i - 1) * b).wait()
            copy(slot, k, i * b).wait()


def kernel(X, W):
    n, m = X.shape
    f = W.shape[1]
    mf = m * f
    b = _ROWS_PER_BLOCK
    wp = jnp.zeros((16, 2 * f), jnp.float32)
    wp = wp.at[:5, :f].set(W).at[8:13, f:].set(W)
    out2d = pl.pallas_call(
        _tc_body,
        grid=(n // b,),
        in_specs=[
            pl.BlockSpec((b, m), lambda i: (i, 0)),
            pl.BlockSpec((16, 2 * f), lambda i: (0, 0)),
        ],
        out_specs=pl.BlockSpec(memory_space=pltpu.MemorySpace.HBM),
        out_shape=jax.ShapeDtypeStruct((n, mf), jnp.float32),
        scratch_shapes=[
            pltpu.VMEM((2, b, mf), jnp.float32),
            pltpu.SemaphoreType.DMA((2, _SPLIT)),
        ],
    )(X.astype(jnp.int32), wp)
    return out2d.reshape(n, m, f)


# speedup vs baseline: 4.2638x; 3.8935x over previous
import jax
import jax.numpy as jnp
from jax.experimental import pallas as pl

_J_PER_BLOCK = 8


def _tc_body(xt_ref, wt8_ref, o_ref):
    jb = xt_ref.shape[0]
    ni = xt_ref.shape[1]
    xt = xt_ref[...].astype(jnp.int32)
    wt8 = wt8_ref[...]
    iota8 = jax.lax.broadcasted_iota(jnp.int32, (8, ni), 0)
    for j in range(jb):
        oh = jnp.where(iota8 == xt[j][None, :], 1.0, 0.0)
        o_ref[j] = jax.lax.dot_general(
            wt8, oh, (((1,), (0,)), ((), ())),
            preferred_element_type=jnp.float32)


def kernel(X, W):
    n, m = X.shape
    f = W.shape[1]
    jb = _J_PER_BLOCK
    xt = X.astype(jnp.int32).T
    wt8 = jnp.zeros((f, 8), jnp.float32).at[:, :5].set(W.T)
    outt = pl.pallas_call(
        _tc_body,
        grid=(m // jb,),
        in_specs=[
            pl.BlockSpec((jb, n), lambda i: (i, 0)),
            pl.BlockSpec((f, 8), lambda i: (0, 0)),
        ],
        out_specs=pl.BlockSpec((jb, f, n), lambda i: (i, 0, 0)),
        out_shape=jax.ShapeDtypeStruct((m, f, n), jnp.float32),
    )(xt, wt8)
    return outt.transpose(2, 0, 1)
